# baseline (device time: 110017 ns/iter reference)
import jax
import jax.numpy as jnp
from jax import lax
from jax.experimental import pallas as pl
from jax.experimental.pallas import tpu as pltpu

K = 2048
D = 2048
F = 8192
M_HALF = D // 2
NRING = 8
FC = F // NRING
NSUB = 4
SC = FC // NSUB
HOPS = 3
P4SC = FC // 2
KH = K // 2


def _ring_coords(q):
    qx = (q >= 4).astype(jnp.int32)
    qz = jnp.where(qx == 0, q, 7 - q)
    return qx, qz


def _body(x_hbm, dy_hbm, out_hbm,
          x_p, x_m, x_stage, dy_v, y_send, y_recv, red_bf, cw_buf,
          ccw_buf, y4_send, y4_recv, red4_bf,
          load_sems, store_sems, y_ssem, y_rsem, y4_ssem, y4_rsem,
          cw_ssem, cw_rsem, ccw_ssem, ccw_rsem):
    my_x = lax.axis_index("x")
    my_y = lax.axis_index("y")
    my_z = lax.axis_index("z")
    p = jnp.where(my_x == 0, my_z, 7 - my_z)
    p4 = (p + 4) % NRING

    rx, rz = _ring_coords((p + 1) % NRING)
    lx, lz = _ring_coords((p - 1) % NRING)
    partner = (my_x, 1 - my_y, my_z)
    right = (rx, my_y, rz)
    left = (lx, my_y, lz)

    xcopies = []
    for i in range(2):
        c = pltpu.make_async_copy(
            x_hbm.at[pl.ds(i * KH, KH)], x_stage.at[i], load_sems.at[i]
        )
        c.start()
        xcopies.append(c)
    dycopy = pltpu.make_async_copy(
        dy_hbm.at[:, pl.ds(p * FC, FC)], dy_v, load_sems.at[2]
    )
    dycopy.start()

    barrier = pltpu.get_barrier_semaphore()
    for nbr in (partner, left, right):
        pl.semaphore_signal(
            barrier, inc=1, device_id=nbr,
            device_id_type=pl.DeviceIdType.MESH,
        )
    pl.semaphore_wait(barrier, 3)

    for i in range(2):
        xcopies[i].wait()

    @pl.when(my_y == 0)
    def _():
        for i in range(2):
            x_p[i * KH:(i + 1) * KH, :] = (
                x_stage[i][:, M_HALF:].astype(jnp.bfloat16))

    @pl.when(my_y == 1)
    def _():
        for i in range(2):
            x_p[i * KH:(i + 1) * KH, :] = (
                x_stage[i][:, :M_HALF].astype(jnp.bfloat16))

    dycopy.wait()

    def half_dot(xref, col_lo, width):
        return lax.dot_general(
            xref[:], dy_v[:, col_lo:col_lo + width].astype(jnp.bfloat16),
            (((0,), (0,)), ((), ())),
            preferred_element_type=jnp.float32,
        )

    y_send[0, :, :] = half_dot(x_p, 0, SC).astype(jnp.bfloat16)
    y_rdmas = [pltpu.make_async_remote_copy(
        src_ref=y_send.at[0], dst_ref=y_recv.at[0],
        send_sem=y_ssem.at[0], recv_sem=y_rsem.at[0],
        device_id=partner, device_id_type=pl.DeviceIdType.MESH,
    )]
    y_rdmas[0].start()

    @pl.when(my_y == 0)
    def _():
        for i in range(2):
            x_m[i * KH:(i + 1) * KH, :] = (
                x_stage[i][:, :M_HALF].astype(jnp.bfloat16))

    @pl.when(my_y == 1)
    def _():
        for i in range(2):
            x_m[i * KH:(i + 1) * KH, :] = (
                x_stage[i][:, M_HALF:].astype(jnp.bfloat16))

    red_bf[0, :, :] = half_dot(x_m, 0, SC).astype(jnp.bfloat16)

    for s in range(1, NSUB):
        y_send[s, :, :] = half_dot(x_p, s * SC, SC).astype(jnp.bfloat16)
        r = pltpu.make_async_remote_copy(
            src_ref=y_send.at[s], dst_ref=y_recv.at[s],
            send_sem=y_ssem.at[s], recv_sem=y_rsem.at[s],
            device_id=partner, device_id_type=pl.DeviceIdType.MESH,
        )
        r.start()
        y_rdmas.append(r)
        red_bf[s, :, :] = half_dot(x_m, s * SC, SC).astype(jnp.bfloat16)

    dy4copy = pltpu.make_async_copy(
        dy_hbm.at[:, pl.ds(p4 * FC, FC)], dy_v, load_sems.at[2]
    )
    dy4copy.start()

    store_jobs = []

    def store(col_start, width, src_ref):
        si = len(store_jobs)
        if si >= 4:
            store_jobs[si - 4].wait()
        st = pltpu.make_async_copy(
            src_ref,
            out_hbm.at[:, pl.ds(col_start, width)],
            store_sems.at[si % 4],
        )
        st.start()
        store_jobs.append(st)

    def hop_rdma(dirn, s, h):
        buf = cw_buf if dirn == "cw" else ccw_buf
        ssem = cw_ssem if dirn == "cw" else ccw_ssem
        rsem = cw_rsem if dirn == "cw" else ccw_rsem
        tgt = right if dirn == "cw" else left
        src = red_bf.at[s] if h == 0 else buf.at[s, h - 1]
        return pltpu.make_async_remote_copy(
            src_ref=src, dst_ref=buf.at[s, h],
            send_sem=ssem.at[s * HOPS + h], recv_sem=rsem.at[s * HOPS + h],
            device_id=tgt, device_id_type=pl.DeviceIdType.MESH,
        )

    rdmas = []
    live = {}
    for s in range(NSUB):
        y_rdmas[s].wait()
        red_bf[s, :, :] = (
            red_bf[s].astype(jnp.float32) + y_recv[s].astype(jnp.float32)
        ).astype(jnp.bfloat16)
        for dirn in ("cw", "ccw"):
            r = hop_rdma(dirn, s, 0)
            r.start()
            rdmas.append(r)
            live[(dirn, s)] = r
        store(p * FC + s * SC, SC, red_bf.at[s])

    y4_rdmas = []
    for h in range(HOPS):
        for s in range(NSUB):
            for dirn in ("cw", "ccw"):
                live[(dirn, s)].wait_recv()
                if h + 1 < HOPS:
                    r = hop_rdma(dirn, s, h + 1)
                    r.start()
                    rdmas.append(r)
                    live[(dirn, s)] = r

        if h == 0:
            dy4copy.wait()
            for hh in range(2):
                y4_send[hh, :, :] = half_dot(
                    x_p, hh * P4SC, P4SC).astype(jnp.bfloat16)
                r4 = pltpu.make_async_remote_copy(
                    src_ref=y4_send.at[hh], dst_ref=y4_recv.at[hh],
                    send_sem=y4_ssem.at[hh], recv_sem=y4_rsem.at[hh],
                    device_id=partner,
                    device_id_type=pl.DeviceIdType.MESH,
                )
                r4.start()
                y4_rdmas.append(r4)
                red4_bf[hh, :, :] = half_dot(
                    x_m, hh * P4SC, P4SC).astype(jnp.bfloat16)

        for s in range(NSUB):
            store(((p - h - 1) % NRING) * FC + s * SC, SC, cw_buf.at[s, h])
            store(((p + h + 1) % NRING) * FC + s * SC, SC, ccw_buf.at[s, h])

    for h in range(2):
        y4_rdmas[h].wait()
        red4_bf[h, :, :] = (
            red4_bf[h].astype(jnp.float32) + y4_recv[h].astype(jnp.float32)
        ).astype(jnp.bfloat16)
        store(p4 * FC + h * P4SC, P4SC, red4_bf.at[h])

    for r in rdmas:
        r.wait_send()
    for st in store_jobs[-4:]:
        st.wait()


def kernel(x, dy):
    return pl.pallas_call(
        _body,
        in_specs=[
            pl.BlockSpec(memory_space=pl.ANY),
            pl.BlockSpec(memory_space=pl.ANY),
        ],
        out_specs=pl.BlockSpec(memory_space=pl.ANY),
        out_shape=jax.ShapeDtypeStruct((M_HALF, F), jnp.bfloat16),
        scratch_shapes=[
            pltpu.VMEM((K, M_HALF), jnp.bfloat16),
            pltpu.VMEM((K, M_HALF), jnp.bfloat16),
            pltpu.VMEM((2, KH, D), jnp.float32),
            pltpu.VMEM((K, FC), jnp.float32),
            pltpu.VMEM((NSUB, M_HALF, SC), jnp.bfloat16),
            pltpu.VMEM((NSUB, M_HALF, SC), jnp.bfloat16),
            pltpu.VMEM((NSUB, M_HALF, SC), jnp.bfloat16),
            pltpu.VMEM((NSUB, HOPS, M_HALF, SC), jnp.bfloat16),
            pltpu.VMEM((NSUB, HOPS, M_HALF, SC), jnp.bfloat16),
            pltpu.VMEM((2, M_HALF, P4SC), jnp.bfloat16),
            pltpu.VMEM((2, M_HALF, P4SC), jnp.bfloat16),
            pltpu.VMEM((2, M_HALF, P4SC), jnp.bfloat16),
            pltpu.SemaphoreType.DMA((3,)),
            pltpu.SemaphoreType.DMA((4,)),
            pltpu.SemaphoreType.DMA((NSUB,)),
            pltpu.SemaphoreType.DMA((NSUB,)),
            pltpu.SemaphoreType.DMA((2,)),
            pltpu.SemaphoreType.DMA((2,)),
            pltpu.SemaphoreType.DMA((NSUB * HOPS,)),
            pltpu.SemaphoreType.DMA((NSUB * HOPS,)),
            pltpu.SemaphoreType.DMA((NSUB * HOPS,)),
            pltpu.SemaphoreType.DMA((NSUB * HOPS,)),
        ],
        compiler_params=pltpu.CompilerParams(
            collective_id=0,
            vmem_limit_bytes=62 * 1024 * 1024,
        ),
    )(x, dy)
